# baseline (speedup 1.0000x reference)
import jax
import jax.numpy as jnp
from jax import lax
from jax.experimental import pallas as pl
from jax.experimental.pallas import tpu as pltpu
from jax.experimental.pallas import tpu_sc as plsc

N_USERS = 15000
N_ITEMS = 35000
N_NODES = 50000
EMB = 64
DECAY = 1e-4
E = 800000
B = 4096

HALF = 25000
DUMP = 25000
ACC_ROWS = 25600
ZROWS = ACC_ROWS // 16
E_PAD = 823296
BLK = 128
NBLK = E_PAD // 16 // BLK
WB = 100


def _mesh():
    return plsc.VectorSubcoreMesh(core_axis_name="c", subcore_axis_name="s")


def _bcast_lane(vec, lane):
    return lax.gather(
        vec, jnp.full((16, 1), lane, jnp.int32),
        lax.GatherDimensionNumbers(
            offset_dims=(), collapsed_slice_dims=(0,),
            start_index_map=(0,)),
        (1,), mode=lax.GatherScatterMode.PROMISE_IN_BOUNDS)


def _hop_body(pk_hbm, x_hbm, out_hbm,
              sd0, sd1, sd2, ix0, ix1, ix2, rows0, rows1, rows2, acc,
              st0, st1, st2, g0, g1, g2, sc0, sc1, sc2):
    c = lax.axis_index("c")
    s = lax.axis_index("s")
    dst_base = c * HALF

    z = jnp.zeros((16,), jnp.float32)

    def zb(i, _):
        for q in range(EMB // 16):
            rows0[i, pl.ds(q * 16, 16)] = z
        return 0
    lax.fori_loop(0, WB, zb, 0)
    for blk in range(ZROWS // WB):
        r0 = s * ZROWS + blk * WB
        pltpu.sync_copy(rows0.at[pl.ds(0, WB)], acc.at[pl.ds(r0, WB)])
    plsc.subcore_barrier()

    row_base = s * NBLK
    sds = (sd0, sd1, sd2)
    ixs = (ix0, ix1, ix2)
    rowss = (rows0, rows1, rows2)
    stsems = (st0, st1, st2)
    gsems = (g0, g1, g2)
    scsems = (sc0, sc1, sc2)

    def cvt_src(sd, ix):
        def cg(g, _):
            v = sd[0, pl.ds(g * 16, 16)]
            ix[0, pl.ds(g * 16, 16)] = v.astype(jnp.int32)
            return 0
        lax.fori_loop(0, BLK // 16, cg, 0)

    pltpu.async_copy(pk_hbm.at[row_base], sd0, st0)
    pltpu.async_copy(pk_hbm.at[row_base + 1], sd1, st1)
    pltpu.make_async_copy(pk_hbm.at[row_base], sd0, st0).wait()
    cvt_src(sd0, ix0)
    pltpu.async_copy(x_hbm.at[ix0.at[0]], rows0, g0)

    def trip(i, _):
        for r in range(3):
            b = i * 3 + r
            rn = (r + 1) % 3
            sd = sds[r]
            ix = ixs[r]
            rows = rowss[r]
            pltpu.make_async_copy(x_hbm.at[ix.at[0]], rows, gsems[r]).wait()

            @pl.when(b + 1 < NBLK)
            def _fire_gather():
                pltpu.make_async_copy(pk_hbm.at[row_base + b + 1],
                                      sds[rn], stsems[rn]).wait()
                cvt_src(sds[rn], ixs[rn])

                @pl.when(b >= 2)
                def _drain_scatter():
                    pltpu.make_async_copy(rowss[rn],
                                          acc.at[ixs[rn].at[1]],
                                          scsems[rn]).wait()
                pltpu.async_copy(x_hbm.at[ixs[rn].at[0]], rowss[rn],
                                 gsems[rn])

            dbf = dst_base.astype(jnp.float32)

            def locg(g, _):
                v = sd[1, pl.ds(g * 16, 16)]
                loc = v - dbf
                ok = (loc >= 0.0) & (loc < float(HALF))
                ix[1, pl.ds(g * 16, 16)] = jnp.where(
                    ok, loc, float(DUMP)).astype(jnp.int32)
                return 0
            lax.fori_loop(0, BLK // 16, locg, 0)

            def grp(g, _):
                wv = sd[2, pl.ds(g * 16, 16)]
                for jj in range(16):
                    e = g * 16 + jj
                    wb_ = _bcast_lane(wv, jj)
                    for qq in range(EMB // 16):
                        rows[e, pl.ds(qq * 16, 16)] = (
                            rows[e, pl.ds(qq * 16, 16)] * wb_)
                return 0
            lax.fori_loop(0, BLK // 16, grp, 0)

            pltpu.async_copy(rows, acc.at[ix.at[1]], scsems[r], add=True)

            @pl.when(b + 2 < NBLK)
            def _fire_stage():
                pltpu.async_copy(pk_hbm.at[row_base + b + 2],
                                 sds[(r + 2) % 3], stsems[(r + 2) % 3])
        return 0
    lax.fori_loop(0, NBLK // 3, trip, 0)
    for r in range(3):
        pltpu.make_async_copy(rowss[r], acc.at[ixs[r].at[1]],
                              scsems[r]).wait()
    plsc.subcore_barrier()

    nwb = jnp.where(s < 15, ZROWS // WB, (HALF - 15 * ZROWS) // WB)

    def wbk(i, _):
        r0 = s * ZROWS + i * WB
        pltpu.sync_copy(acc.at[pl.ds(r0, WB)], rows0.at[pl.ds(0, WB)])
        pltpu.sync_copy(rows0.at[pl.ds(0, WB)],
                        out_hbm.at[pl.ds(dst_base + r0, WB)])
        return 0
    lax.fori_loop(0, nwb, wbk, 0)


def _make_hop():
    return pl.kernel(
        _hop_body,
        out_type=jax.ShapeDtypeStruct((N_NODES, EMB), jnp.float32),
        mesh=_mesh(),
        compiler_params=pltpu.CompilerParams(use_tc_tiling_on_sc=False),
        scratch_types=(
            [pltpu.VMEM((3, 128), jnp.float32)] * 3
            + [pltpu.VMEM((2, 128), jnp.int32)] * 3
            + [pltpu.VMEM((BLK, EMB), jnp.float32)] * 3
            + [pltpu.VMEM_SHARED((ACC_ROWS, EMB), jnp.float32)]
            + [pltpu.SemaphoreType.DMA] * 9
        ),
    )


def _gather_body(x0, x1, x2, x3, u_hbm, p_hbm, n_hbm,
                 ue_o, pe_o, ne_o, u0_o, p0_o, n0_o,
                 idx_v, rows, accb, sem):
    c = lax.axis_index("c")
    s = lax.axis_index("s")
    base = (c * 16 + s) * (B // 32)
    nrow = B // 32

    for (src_idx, off, out_m, out_0) in ((u_hbm, 0, ue_o, u0_o),
                                         (p_hbm, N_USERS, pe_o, p0_o),
                                         (n_hbm, N_USERS, ne_o, n0_o)):
        pltpu.sync_copy(src_idx.at[pl.ds(base, nrow)], idx_v)
        if off:
            for g in range(nrow // 16):
                idx_v[pl.ds(g * 16, 16)] = idx_v[pl.ds(g * 16, 16)] + off
        pltpu.async_copy(x0.at[idx_v], rows, sem).wait()
        pltpu.sync_copy(rows, out_0.at[pl.ds(base, nrow)])

        def cp(r, _):
            for q in range(EMB // 16):
                accb[r, pl.ds(q * 16, 16)] = rows[r, pl.ds(q * 16, 16)]
            return 0
        lax.fori_loop(0, nrow, cp, 0)
        for t in (x1, x2, x3):
            pltpu.async_copy(t.at[idx_v], rows, sem).wait()

            def addr(r, _):
                for q in range(EMB // 16):
                    accb[r, pl.ds(q * 16, 16)] = (accb[r, pl.ds(q * 16, 16)]
                                                  + rows[r, pl.ds(q * 16, 16)])
                return 0
            lax.fori_loop(0, nrow, addr, 0)

        def mn(r, _):
            for q in range(EMB // 16):
                accb[r, pl.ds(q * 16, 16)] = accb[r, pl.ds(q * 16, 16)] * 0.25
            return 0
        lax.fori_loop(0, nrow, mn, 0)
        pltpu.sync_copy(accb, out_m.at[pl.ds(base, nrow)])


def _make_gather():
    sds = jax.ShapeDtypeStruct((B, EMB), jnp.float32)
    return pl.kernel(
        _gather_body,
        out_type=(sds,) * 6,
        mesh=_mesh(),
        compiler_params=pltpu.CompilerParams(use_tc_tiling_on_sc=False),
        scratch_types=[
            pltpu.VMEM((B // 32,), jnp.int32),
            pltpu.VMEM((B // 32, EMB), jnp.float32),
            pltpu.VMEM((B // 32, EMB), jnp.float32),
            pltpu.SemaphoreType.DMA,
        ],
    )


def _loss_body(ue, pe, ne, u0, p0, n0, out_ref):
    uev = ue[...]
    pos_s = jnp.sum(uev * pe[...], axis=1)
    neg_s = jnp.sum(uev * ne[...], axis=1)
    mf = jnp.mean(jnp.log(1.0 + jnp.exp(neg_s - pos_s)))
    reg = (jnp.sum(u0[...] ** 2) + jnp.sum(p0[...] ** 2)
           + jnp.sum(n0[...] ** 2)) * 0.5
    emb = jnp.float32(DECAY) * reg / B
    lanes = lax.broadcasted_iota(jnp.int32, (1, 128), 1)
    row = jnp.where(lanes == 0, mf + emb,
                    jnp.where(lanes == 1, mf,
                              jnp.where(lanes == 2, emb, 0.0)))
    out_ref[...] = row.astype(jnp.float32)


_loss = pl.pallas_call(
    _loss_body,
    out_shape=jax.ShapeDtypeStruct((1, 128), jnp.float32),
)

_hop = _make_hop()
_gather = _make_gather()


def kernel(users, pos_items, neg_items, adj_indices, adj_values,
           user_embed, item_embed):
    users = users.astype(jnp.int32)
    pos = pos_items.astype(jnp.int32)
    neg = neg_items[:, 0].astype(jnp.int32)
    dst = adj_indices[0].astype(jnp.int32)
    src = adj_indices[1].astype(jnp.int32)
    w = adj_values.astype(jnp.float32)

    pad = E_PAD - E
    srcp = jnp.concatenate([src.astype(jnp.float32),
                            jnp.zeros((pad,), jnp.float32)])
    dstp = jnp.concatenate([dst.astype(jnp.float32),
                            jnp.full((pad,), float(2 ** 25), jnp.float32)])
    wp = jnp.concatenate([w, jnp.zeros((pad,), jnp.float32)])
    pk = jnp.stack([srcp.reshape(E_PAD // 128, 128),
                    dstp.reshape(E_PAD // 128, 128),
                    wp.reshape(E_PAD // 128, 128)], axis=1)

    x0 = jnp.concatenate([user_embed.astype(jnp.float32),
                          item_embed.astype(jnp.float32)], axis=0)
    x1 = _hop(pk, x0)
    x2 = _hop(pk, x1)
    x3 = _hop(pk, x2)

    ue, pe, ne, u0, p0, n0 = _gather(x0, x1, x2, x3, users, pos, neg)
    row = _loss(ue, pe, ne, u0, p0, n0)
    return (row[0, 0], row[0, 1], row[0, 2])


# speedup vs baseline: 1.0001x; 1.0001x over previous
import jax
import jax.numpy as jnp
from jax import lax
from jax.experimental import pallas as pl
from jax.experimental.pallas import tpu as pltpu
from jax.experimental.pallas import tpu_sc as plsc

N_USERS = 15000
N_ITEMS = 35000
N_NODES = 50000
EMB = 64
DECAY = 1e-4
E = 800000
B = 4096

HALF = 25000
DUMP = 25000
ACC_ROWS = 25600
ZROWS = ACC_ROWS // 16
E_PAD = 823296
BLK = 128
NBLK = E_PAD // 16 // BLK
WB = 100


def _mesh():
    return plsc.VectorSubcoreMesh(core_axis_name="c", subcore_axis_name="s")


def _bcast_lane(vec, lane):
    return lax.gather(
        vec, jnp.full((16, 1), lane, jnp.int32),
        lax.GatherDimensionNumbers(
            offset_dims=(), collapsed_slice_dims=(0,),
            start_index_map=(0,)),
        (1,), mode=lax.GatherScatterMode.PROMISE_IN_BOUNDS)


def _hop_body(pk_hbm, x_hbm, out_hbm,
              sd0, sd1, sd2, ix0, ix1, ix2, rows0, rows1, rows2, acc,
              st0, st1, st2, g0, g1, g2, h0, h1, h2, sc0, sc1, sc2):
    c = lax.axis_index("c")
    s = lax.axis_index("s")
    dst_base = c * HALF

    z = jnp.zeros((16,), jnp.float32)

    def zb(i, _):
        for q in range(EMB // 16):
            rows0[i, pl.ds(q * 16, 16)] = z
        return 0
    lax.fori_loop(0, WB, zb, 0)
    for blk in range(ZROWS // WB):
        r0 = s * ZROWS + blk * WB
        pltpu.sync_copy(rows0.at[pl.ds(0, WB)], acc.at[pl.ds(r0, WB)])
    plsc.subcore_barrier()

    row_base = s * NBLK
    sds = (sd0, sd1, sd2)
    ixs = (ix0, ix1, ix2)
    rowss = (rows0, rows1, rows2)
    stsems = (st0, st1, st2)
    gsems = (g0, g1, g2)
    hsems = (h0, h1, h2)
    scsems = (sc0, sc1, sc2)

    def cvt_src(sd, ix):
        def cg(g, _):
            v = sd[0, pl.ds(g * 16, 16)]
            ix[0, pl.ds(g * 16, 16)] = v.astype(jnp.int32)
            return 0
        lax.fori_loop(0, BLK // 16, cg, 0)

    def fire_gather(ix, rows, semA, semB):
        pltpu.async_copy(x_hbm.at[ix.at[0, pl.ds(0, BLK // 2)]],
                         rows.at[pl.ds(0, BLK // 2)], semA)
        pltpu.async_copy(x_hbm.at[ix.at[0, pl.ds(BLK // 2, BLK // 2)]],
                         rows.at[pl.ds(BLK // 2, BLK // 2)], semB)

    def wait_gather(ix, rows, semA, semB):
        pltpu.make_async_copy(x_hbm.at[ix.at[0, pl.ds(0, BLK // 2)]],
                              rows.at[pl.ds(0, BLK // 2)], semA).wait()
        pltpu.make_async_copy(x_hbm.at[ix.at[0, pl.ds(BLK // 2, BLK // 2)]],
                              rows.at[pl.ds(BLK // 2, BLK // 2)], semB).wait()

    pltpu.async_copy(pk_hbm.at[row_base], sd0, st0)
    pltpu.async_copy(pk_hbm.at[row_base + 1], sd1, st1)
    pltpu.make_async_copy(pk_hbm.at[row_base], sd0, st0).wait()
    cvt_src(sd0, ix0)
    fire_gather(ix0, rows0, g0, h0)

    def trip(i, _):
        for r in range(3):
            b = i * 3 + r
            rn = (r + 1) % 3
            sd = sds[r]
            ix = ixs[r]
            rows = rowss[r]
            wait_gather(ix, rows, gsems[r], hsems[r])

            @pl.when(b + 1 < NBLK)
            def _fire_gather():
                pltpu.make_async_copy(pk_hbm.at[row_base + b + 1],
                                      sds[rn], stsems[rn]).wait()
                cvt_src(sds[rn], ixs[rn])

                @pl.when(b >= 2)
                def _drain_scatter():
                    pltpu.make_async_copy(rowss[rn],
                                          acc.at[ixs[rn].at[1]],
                                          scsems[rn]).wait()
                fire_gather(ixs[rn], rowss[rn], gsems[rn], hsems[rn])

            dbf = dst_base.astype(jnp.float32)

            def locg(g, _):
                v = sd[1, pl.ds(g * 16, 16)]
                loc = v - dbf
                ok = (loc >= 0.0) & (loc < float(HALF))
                ix[1, pl.ds(g * 16, 16)] = jnp.where(
                    ok, loc, float(DUMP)).astype(jnp.int32)
                return 0
            lax.fori_loop(0, BLK // 16, locg, 0)

            def grp(g, _):
                wv = sd[2, pl.ds(g * 16, 16)]
                for jj in range(16):
                    e = g * 16 + jj
                    wb_ = _bcast_lane(wv, jj)
                    for qq in range(EMB // 16):
                        rows[e, pl.ds(qq * 16, 16)] = (
                            rows[e, pl.ds(qq * 16, 16)] * wb_)
                return 0
            lax.fori_loop(0, BLK // 16, grp, 0)

            pltpu.async_copy(rows, acc.at[ix.at[1]], scsems[r], add=True)

            @pl.when(b + 2 < NBLK)
            def _fire_stage():
                pltpu.async_copy(pk_hbm.at[row_base + b + 2],
                                 sds[(r + 2) % 3], stsems[(r + 2) % 3])
        return 0
    lax.fori_loop(0, NBLK // 3, trip, 0)
    for r in range(3):
        pltpu.make_async_copy(rowss[r], acc.at[ixs[r].at[1]],
                              scsems[r]).wait()
    plsc.subcore_barrier()

    nwb = jnp.where(s < 15, ZROWS // WB, (HALF - 15 * ZROWS) // WB)

    def wbk(i, _):
        r0 = s * ZROWS + i * WB
        pltpu.sync_copy(acc.at[pl.ds(r0, WB)], rows0.at[pl.ds(0, WB)])
        pltpu.sync_copy(rows0.at[pl.ds(0, WB)],
                        out_hbm.at[pl.ds(dst_base + r0, WB)])
        return 0
    lax.fori_loop(0, nwb, wbk, 0)


def _make_hop():
    return pl.kernel(
        _hop_body,
        out_type=jax.ShapeDtypeStruct((N_NODES, EMB), jnp.float32),
        mesh=_mesh(),
        compiler_params=pltpu.CompilerParams(use_tc_tiling_on_sc=False),
        scratch_types=(
            [pltpu.VMEM((3, 128), jnp.float32)] * 3
            + [pltpu.VMEM((2, 128), jnp.int32)] * 3
            + [pltpu.VMEM((BLK, EMB), jnp.float32)] * 3
            + [pltpu.VMEM_SHARED((ACC_ROWS, EMB), jnp.float32)]
            + [pltpu.SemaphoreType.DMA] * 12
        ),
    )


def _gather_body(x0, x1, x2, x3, u_hbm, p_hbm, n_hbm,
                 ue_o, pe_o, ne_o, u0_o, p0_o, n0_o,
                 idx_v, rows, accb, sem):
    c = lax.axis_index("c")
    s = lax.axis_index("s")
    base = (c * 16 + s) * (B // 32)
    nrow = B // 32

    for (src_idx, off, out_m, out_0) in ((u_hbm, 0, ue_o, u0_o),
                                         (p_hbm, N_USERS, pe_o, p0_o),
                                         (n_hbm, N_USERS, ne_o, n0_o)):
        pltpu.sync_copy(src_idx.at[pl.ds(base, nrow)], idx_v)
        if off:
            for g in range(nrow // 16):
                idx_v[pl.ds(g * 16, 16)] = idx_v[pl.ds(g * 16, 16)] + off
        pltpu.async_copy(x0.at[idx_v], rows, sem).wait()
        pltpu.sync_copy(rows, out_0.at[pl.ds(base, nrow)])

        def cp(r, _):
            for q in range(EMB // 16):
                accb[r, pl.ds(q * 16, 16)] = rows[r, pl.ds(q * 16, 16)]
            return 0
        lax.fori_loop(0, nrow, cp, 0)
        for t in (x1, x2, x3):
            pltpu.async_copy(t.at[idx_v], rows, sem).wait()

            def addr(r, _):
                for q in range(EMB // 16):
                    accb[r, pl.ds(q * 16, 16)] = (accb[r, pl.ds(q * 16, 16)]
                                                  + rows[r, pl.ds(q * 16, 16)])
                return 0
            lax.fori_loop(0, nrow, addr, 0)

        def mn(r, _):
            for q in range(EMB // 16):
                accb[r, pl.ds(q * 16, 16)] = accb[r, pl.ds(q * 16, 16)] * 0.25
            return 0
        lax.fori_loop(0, nrow, mn, 0)
        pltpu.sync_copy(accb, out_m.at[pl.ds(base, nrow)])


def _make_gather():
    sds = jax.ShapeDtypeStruct((B, EMB), jnp.float32)
    return pl.kernel(
        _gather_body,
        out_type=(sds,) * 6,
        mesh=_mesh(),
        compiler_params=pltpu.CompilerParams(use_tc_tiling_on_sc=False),
        scratch_types=[
            pltpu.VMEM((B // 32,), jnp.int32),
            pltpu.VMEM((B // 32, EMB), jnp.float32),
            pltpu.VMEM((B // 32, EMB), jnp.float32),
            pltpu.SemaphoreType.DMA,
        ],
    )


def _loss_body(ue, pe, ne, u0, p0, n0, out_ref):
    uev = ue[...]
    pos_s = jnp.sum(uev * pe[...], axis=1)
    neg_s = jnp.sum(uev * ne[...], axis=1)
    mf = jnp.mean(jnp.log(1.0 + jnp.exp(neg_s - pos_s)))
    reg = (jnp.sum(u0[...] ** 2) + jnp.sum(p0[...] ** 2)
           + jnp.sum(n0[...] ** 2)) * 0.5
    emb = jnp.float32(DECAY) * reg / B
    lanes = lax.broadcasted_iota(jnp.int32, (1, 128), 1)
    row = jnp.where(lanes == 0, mf + emb,
                    jnp.where(lanes == 1, mf,
                              jnp.where(lanes == 2, emb, 0.0)))
    out_ref[...] = row.astype(jnp.float32)


_loss = pl.pallas_call(
    _loss_body,
    out_shape=jax.ShapeDtypeStruct((1, 128), jnp.float32),
)

_hop = _make_hop()
_gather = _make_gather()


def kernel(users, pos_items, neg_items, adj_indices, adj_values,
           user_embed, item_embed):
    users = users.astype(jnp.int32)
    pos = pos_items.astype(jnp.int32)
    neg = neg_items[:, 0].astype(jnp.int32)
    dst = adj_indices[0].astype(jnp.int32)
    src = adj_indices[1].astype(jnp.int32)
    w = adj_values.astype(jnp.float32)

    pad = E_PAD - E
    srcp = jnp.concatenate([src.astype(jnp.float32),
                            jnp.zeros((pad,), jnp.float32)])
    dstp = jnp.concatenate([dst.astype(jnp.float32),
                            jnp.full((pad,), float(2 ** 25), jnp.float32)])
    wp = jnp.concatenate([w, jnp.zeros((pad,), jnp.float32)])
    pk = jnp.stack([srcp.reshape(E_PAD // 128, 128),
                    dstp.reshape(E_PAD // 128, 128),
                    wp.reshape(E_PAD // 128, 128)], axis=1)

    x0 = jnp.concatenate([user_embed.astype(jnp.float32),
                          item_embed.astype(jnp.float32)], axis=0)
    x1 = _hop(pk, x0)
    x2 = _hop(pk, x1)
    x3 = _hop(pk, x2)

    ue, pe, ne, u0, p0, n0 = _gather(x0, x1, x2, x3, users, pos, neg)
    row = _loss(ue, pe, ne, u0, p0, n0)
    return (row[0, 0], row[0, 1], row[0, 2])


# speedup vs baseline: 1.2087x; 1.2086x over previous
import jax
import jax.numpy as jnp
from jax import lax
from jax.experimental import pallas as pl
from jax.experimental.pallas import tpu as pltpu
from jax.experimental.pallas import tpu_sc as plsc

N_USERS = 15000
N_ITEMS = 35000
N_NODES = 50000
EMB = 64
DECAY = 1e-4
E = 800000
B = 4096

HALF = 25000
DUMP = 25000
ACC_ROWS = 25600
ZROWS = ACC_ROWS // 16
E_PAD = 823296
BLK = 128
NBLK = E_PAD // 16 // BLK
WB = 100


def _mesh():
    return plsc.VectorSubcoreMesh(core_axis_name="c", subcore_axis_name="s")


def _bcast_lane(vec, lane):
    return lax.gather(
        vec, jnp.full((16, 1), lane, jnp.int32),
        lax.GatherDimensionNumbers(
            offset_dims=(), collapsed_slice_dims=(0,),
            start_index_map=(0,)),
        (1,), mode=lax.GatherScatterMode.PROMISE_IN_BOUNDS)


def _hop_body(pk_hbm, x_hbm, out_hbm,
              sd0, sd1, sd2, ix0, ix1, ix2, rows0, rows1, rows2, acc,
              st0, st1, st2, g0, g1, g2, h0, h1, h2, sc0, sc1, sc2):
    c = lax.axis_index("c")
    s = lax.axis_index("s")
    dst_base = c * HALF

    z = jnp.zeros((16,), jnp.float32)

    def zb(i, _):
        for q in range(EMB // 16):
            rows0[i, pl.ds(q * 16, 16)] = z
        return 0
    lax.fori_loop(0, WB, zb, 0)
    for blk in range(ZROWS // WB):
        r0 = s * ZROWS + blk * WB
        pltpu.sync_copy(rows0.at[pl.ds(0, WB)], acc.at[pl.ds(r0, WB)])
    plsc.subcore_barrier()

    row_base = s * NBLK
    sds = (sd0, sd1, sd2)
    ixs = (ix0, ix1, ix2)
    rowss = (rows0, rows1, rows2)
    stsems = (st0, st1, st2)
    gsems = (g0, g1, g2)
    hsems = (h0, h1, h2)
    scsems = (sc0, sc1, sc2)

    def cvt_src(sd, ix):
        @plsc.parallel_loop(0, BLK // 16, unroll=4)
        def _cg(g):
            v = sd[0, pl.ds(g * 16, 16)]
            ix[0, pl.ds(g * 16, 16)] = v.astype(jnp.int32)

    def fire_gather(ix, rows, semA, semB):
        pltpu.async_copy(x_hbm.at[ix.at[0, pl.ds(0, BLK // 2)]],
                         rows.at[pl.ds(0, BLK // 2)], semA)
        pltpu.async_copy(x_hbm.at[ix.at[0, pl.ds(BLK // 2, BLK // 2)]],
                         rows.at[pl.ds(BLK // 2, BLK // 2)], semB)

    def wait_gather(ix, rows, semA, semB):
        pltpu.make_async_copy(x_hbm.at[ix.at[0, pl.ds(0, BLK // 2)]],
                              rows.at[pl.ds(0, BLK // 2)], semA).wait()
        pltpu.make_async_copy(x_hbm.at[ix.at[0, pl.ds(BLK // 2, BLK // 2)]],
                              rows.at[pl.ds(BLK // 2, BLK // 2)], semB).wait()

    pltpu.async_copy(pk_hbm.at[row_base], sd0, st0)
    pltpu.async_copy(pk_hbm.at[row_base + 1], sd1, st1)
    pltpu.make_async_copy(pk_hbm.at[row_base], sd0, st0).wait()
    cvt_src(sd0, ix0)
    fire_gather(ix0, rows0, g0, h0)

    def trip(i, _):
        for r in range(3):
            b = i * 3 + r
            rn = (r + 1) % 3
            sd = sds[r]
            ix = ixs[r]
            rows = rowss[r]
            wait_gather(ix, rows, gsems[r], hsems[r])

            @pl.when(b + 1 < NBLK)
            def _fire_gather():
                pltpu.make_async_copy(pk_hbm.at[row_base + b + 1],
                                      sds[rn], stsems[rn]).wait()
                cvt_src(sds[rn], ixs[rn])

                @pl.when(b >= 2)
                def _drain_scatter():
                    pltpu.make_async_copy(rowss[rn],
                                          acc.at[ixs[rn].at[1]],
                                          scsems[rn]).wait()
                fire_gather(ixs[rn], rowss[rn], gsems[rn], hsems[rn])

            dbf = dst_base.astype(jnp.float32)

            @plsc.parallel_loop(0, BLK // 16, unroll=4)
            def _locg(g):
                v = sd[1, pl.ds(g * 16, 16)]
                loc = v - dbf
                ok = (loc >= 0.0) & (loc < float(HALF))
                ix[1, pl.ds(g * 16, 16)] = jnp.where(
                    ok, loc, float(DUMP)).astype(jnp.int32)

            @plsc.parallel_loop(0, BLK // 16, unroll=2)
            def _grp(g):
                wv = sd[2, pl.ds(g * 16, 16)]
                for jj in range(16):
                    e = g * 16 + jj
                    wb_ = _bcast_lane(wv, jj)
                    for qq in range(EMB // 16):
                        rows[e, pl.ds(qq * 16, 16)] = (
                            rows[e, pl.ds(qq * 16, 16)] * wb_)

            pltpu.async_copy(rows, acc.at[ix.at[1]], scsems[r], add=True)

            @pl.when(b + 2 < NBLK)
            def _fire_stage():
                pltpu.async_copy(pk_hbm.at[row_base + b + 2],
                                 sds[(r + 2) % 3], stsems[(r + 2) % 3])
        return 0
    lax.fori_loop(0, NBLK // 3, trip, 0)
    for r in range(3):
        pltpu.make_async_copy(rowss[r], acc.at[ixs[r].at[1]],
                              scsems[r]).wait()
    plsc.subcore_barrier()

    nwb = jnp.where(s < 15, ZROWS // WB, (HALF - 15 * ZROWS) // WB)

    def wbk(i, _):
        r0 = s * ZROWS + i * WB
        pltpu.sync_copy(acc.at[pl.ds(r0, WB)], rows0.at[pl.ds(0, WB)])
        pltpu.sync_copy(rows0.at[pl.ds(0, WB)],
                        out_hbm.at[pl.ds(dst_base + r0, WB)])
        return 0
    lax.fori_loop(0, nwb, wbk, 0)


def _make_hop():
    return pl.kernel(
        _hop_body,
        out_type=jax.ShapeDtypeStruct((N_NODES, EMB), jnp.float32),
        mesh=_mesh(),
        compiler_params=pltpu.CompilerParams(use_tc_tiling_on_sc=False),
        scratch_types=(
            [pltpu.VMEM((3, 128), jnp.float32)] * 3
            + [pltpu.VMEM((2, 128), jnp.int32)] * 3
            + [pltpu.VMEM((BLK, EMB), jnp.float32)] * 3
            + [pltpu.VMEM_SHARED((ACC_ROWS, EMB), jnp.float32)]
            + [pltpu.SemaphoreType.DMA] * 12
        ),
    )


def _gather_body(x0, x1, x2, x3, u_hbm, p_hbm, n_hbm,
                 ue_o, pe_o, ne_o, u0_o, p0_o, n0_o,
                 idx_v, rows, accb, sem):
    c = lax.axis_index("c")
    s = lax.axis_index("s")
    base = (c * 16 + s) * (B // 32)
    nrow = B // 32

    for (src_idx, off, out_m, out_0) in ((u_hbm, 0, ue_o, u0_o),
                                         (p_hbm, N_USERS, pe_o, p0_o),
                                         (n_hbm, N_USERS, ne_o, n0_o)):
        pltpu.sync_copy(src_idx.at[pl.ds(base, nrow)], idx_v)
        if off:
            for g in range(nrow // 16):
                idx_v[pl.ds(g * 16, 16)] = idx_v[pl.ds(g * 16, 16)] + off
        pltpu.async_copy(x0.at[idx_v], rows, sem).wait()
        pltpu.sync_copy(rows, out_0.at[pl.ds(base, nrow)])

        def cp(r, _):
            for q in range(EMB // 16):
                accb[r, pl.ds(q * 16, 16)] = rows[r, pl.ds(q * 16, 16)]
            return 0
        lax.fori_loop(0, nrow, cp, 0)
        for t in (x1, x2, x3):
            pltpu.async_copy(t.at[idx_v], rows, sem).wait()

            def addr(r, _):
                for q in range(EMB // 16):
                    accb[r, pl.ds(q * 16, 16)] = (accb[r, pl.ds(q * 16, 16)]
                                                  + rows[r, pl.ds(q * 16, 16)])
                return 0
            lax.fori_loop(0, nrow, addr, 0)

        def mn(r, _):
            for q in range(EMB // 16):
                accb[r, pl.ds(q * 16, 16)] = accb[r, pl.ds(q * 16, 16)] * 0.25
            return 0
        lax.fori_loop(0, nrow, mn, 0)
        pltpu.sync_copy(accb, out_m.at[pl.ds(base, nrow)])


def _make_gather():
    sds = jax.ShapeDtypeStruct((B, EMB), jnp.float32)
    return pl.kernel(
        _gather_body,
        out_type=(sds,) * 6,
        mesh=_mesh(),
        compiler_params=pltpu.CompilerParams(use_tc_tiling_on_sc=False),
        scratch_types=[
            pltpu.VMEM((B // 32,), jnp.int32),
            pltpu.VMEM((B // 32, EMB), jnp.float32),
            pltpu.VMEM((B // 32, EMB), jnp.float32),
            pltpu.SemaphoreType.DMA,
        ],
    )


def _loss_body(ue, pe, ne, u0, p0, n0, out_ref):
    uev = ue[...]
    pos_s = jnp.sum(uev * pe[...], axis=1)
    neg_s = jnp.sum(uev * ne[...], axis=1)
    mf = jnp.mean(jnp.log(1.0 + jnp.exp(neg_s - pos_s)))
    reg = (jnp.sum(u0[...] ** 2) + jnp.sum(p0[...] ** 2)
           + jnp.sum(n0[...] ** 2)) * 0.5
    emb = jnp.float32(DECAY) * reg / B
    lanes = lax.broadcasted_iota(jnp.int32, (1, 128), 1)
    row = jnp.where(lanes == 0, mf + emb,
                    jnp.where(lanes == 1, mf,
                              jnp.where(lanes == 2, emb, 0.0)))
    out_ref[...] = row.astype(jnp.float32)


_loss = pl.pallas_call(
    _loss_body,
    out_shape=jax.ShapeDtypeStruct((1, 128), jnp.float32),
)

_hop = _make_hop()
_gather = _make_gather()


def kernel(users, pos_items, neg_items, adj_indices, adj_values,
           user_embed, item_embed):
    users = users.astype(jnp.int32)
    pos = pos_items.astype(jnp.int32)
    neg = neg_items[:, 0].astype(jnp.int32)
    dst = adj_indices[0].astype(jnp.int32)
    src = adj_indices[1].astype(jnp.int32)
    w = adj_values.astype(jnp.float32)

    pad = E_PAD - E
    srcp = jnp.concatenate([src.astype(jnp.float32),
                            jnp.zeros((pad,), jnp.float32)])
    dstp = jnp.concatenate([dst.astype(jnp.float32),
                            jnp.full((pad,), float(2 ** 25), jnp.float32)])
    wp = jnp.concatenate([w, jnp.zeros((pad,), jnp.float32)])
    pk = jnp.stack([srcp.reshape(E_PAD // 128, 128),
                    dstp.reshape(E_PAD // 128, 128),
                    wp.reshape(E_PAD // 128, 128)], axis=1)

    x0 = jnp.concatenate([user_embed.astype(jnp.float32),
                          item_embed.astype(jnp.float32)], axis=0)
    x1 = _hop(pk, x0)
    x2 = _hop(pk, x1)
    x3 = _hop(pk, x2)

    ue, pe, ne, u0, p0, n0 = _gather(x0, x1, x2, x3, users, pos, neg)
    row = _loss(ue, pe, ne, u0, p0, n0)
    return (row[0, 0], row[0, 1], row[0, 2])
